# speedup vs baseline: 5.7336x; 5.7336x over previous
import functools

import jax
import jax.numpy as jnp
from jax.experimental import pallas as pl
from jax.experimental.pallas import tpu as pltpu

E = 8
K = 2
D = 1024
H = 4096
HT = 1024
NH = H // HT


def _gelu_exact(v):
    return v * 0.5 * (1.0 + jax.lax.erf(v * (2.0 ** -0.5)))


def _moe_dense_body(x_ref, wg_ref, w1_ref, w2_ref, out_ref, wfull_s):
    e = pl.program_id(0)
    h = pl.program_id(1)

    @pl.when(jnp.logical_and(e == 0, h == 0))
    def _router():
        xv = x_ref[...]
        wg = wg_ref[0:E, :]
        logits = jax.lax.dot_general(
            xv, wg, (((1,), (1,)), ((), ())),
            preferred_element_type=jnp.float32)
        m = jnp.max(logits, axis=1, keepdims=True)
        p = jnp.exp(logits - m)
        p = p / jnp.sum(p, axis=1, keepdims=True)
        cols = jax.lax.broadcasted_iota(jnp.int32, p.shape, 1)
        m1 = jnp.max(p, axis=1, keepdims=True)
        i1 = jnp.min(jnp.where(p == m1, cols, E), axis=1, keepdims=True)
        mask1 = cols == i1
        p2 = jnp.where(mask1, -1.0, p)
        m2 = jnp.max(p2, axis=1, keepdims=True)
        i2 = jnp.min(jnp.where(p2 == m2, cols, E), axis=1, keepdims=True)
        wfull_s[...] = jnp.where(mask1 | (cols == i2), p, 0.0)

    xv = x_ref[...]
    hblk = _gelu_exact(
        jax.lax.dot_general(xv, w1_ref[0], (((1,), (0,)), ((), ())),
                            preferred_element_type=jnp.float32))
    part = jax.lax.dot_general(hblk, w2_ref[0], (((1,), (0,)), ((), ())),
                               preferred_element_type=jnp.float32)
    cols = jax.lax.broadcasted_iota(jnp.int32, wfull_s.shape, 1)
    we = jnp.sum(jnp.where(cols == e, wfull_s[...], 0.0), axis=1,
                 keepdims=True)
    contrib = part * we

    @pl.when(jnp.logical_and(e == 0, h == 0))
    def _init():
        out_ref[...] = contrib

    @pl.when(jnp.logical_not(jnp.logical_and(e == 0, h == 0)))
    def _acc():
        out_ref[...] = out_ref[...] + contrib


@functools.partial(jax.jit, static_argnames=("interpret",))
def _moe_dense(xf, Wg, W1, W2, interpret=False):
    N = xf.shape[0]
    return pl.pallas_call(
        _moe_dense_body,
        grid=(E, NH),
        in_specs=[
            pl.BlockSpec((N, D), lambda e, h: (0, 0)),
            pl.BlockSpec((2 * E, D), lambda e, h: (0, 0)),
            pl.BlockSpec((1, D, HT), lambda e, h: (e, 0, h)),
            pl.BlockSpec((1, HT, D), lambda e, h: (e, h, 0)),
        ],
        out_specs=pl.BlockSpec((N, D), lambda e, h: (0, 0)),
        out_shape=jax.ShapeDtypeStruct((N, D), jnp.float32),
        scratch_shapes=[pltpu.VMEM((N, E), jnp.float32)],
        interpret=interpret,
    )(xf, Wg, W1, W2)


def kernel(x, Wg, W1, W2):
    Bs, Ss, Dm = x.shape
    xf = x.reshape(-1, Dm)
    y = _moe_dense(xf, Wg, W1, W2)
    return y.reshape(Bs, Ss, Dm)
